# baseline (speedup 1.0000x reference)
import functools

import jax
import jax.numpy as jnp
from jax import lax
from jax.experimental import pallas as pl
from jax.experimental.pallas import tpu as pltpu
from jax.experimental.pallas import tpu_sc as plsc

B = 16384
D = 64
L = 16
NC = 2
NS = 16
NW = NC * NS
BPW = B // NW
CHUNK = 128
NCHUNK = BPW // CHUNK

_mesh = plsc.VectorSubcoreMesh(
    core_axis_name="c", subcore_axis_name="s", num_cores=NC, num_subcores=NS
)


@functools.partial(
    pl.kernel,
    out_type=jax.ShapeDtypeStruct((B,), jnp.float32),
    mesh=_mesh,
    compiler_params=pltpu.CompilerParams(
        needs_layout_passes=False, use_tc_tiling_on_sc=False
    ),
    scratch_types=[
        pltpu.VMEM((NCHUNK, CHUNK), jnp.int32),
        pltpu.VMEM((NCHUNK, CHUNK), jnp.int32),
        pltpu.VMEM((BPW, D), jnp.float32),
        pltpu.VMEM((BPW, D), jnp.float32),
        pltpu.VMEM((BPW * L,), jnp.float32),
        pltpu.VMEM((BPW,), jnp.float32),
        pltpu.SemaphoreType.DMA,
    ],
)
def _line_first_sc(i_hbm, j_hbm, emb_hbm, out_hbm,
                   idx_i, idx_j, rows_i, rows_j, q_v, out_v, sem):
    wid = lax.axis_index("s") * NC + lax.axis_index("c")
    base = wid * BPW

    for c in range(NCHUNK):
        pltpu.sync_copy(i_hbm.at[pl.ds(base + c * CHUNK, CHUNK)], idx_i.at[c])
        pltpu.sync_copy(j_hbm.at[pl.ds(base + c * CHUNK, CHUNK)], idx_j.at[c])

    copies = []
    for c in range(NCHUNK):
        copies.append(pltpu.async_copy(
            emb_hbm.at[idx_i.at[c]], rows_i.at[pl.ds(c * CHUNK, CHUNK)], sem))
        copies.append(pltpu.async_copy(
            emb_hbm.at[idx_j.at[c]], rows_j.at[pl.ds(c * CHUNK, CHUNK)], sem))
    for cp in copies:
        cp.wait()

    def row_body(b, carry):
        acc = rows_i[b, pl.ds(0, L)] * rows_j[b, pl.ds(0, L)]
        for c in range(1, D // L):
            acc = acc + rows_i[b, pl.ds(c * L, L)] * rows_j[b, pl.ds(c * L, L)]
        q_v[pl.ds(b * L, L)] = acc
        return carry

    lax.fori_loop(0, BPW, row_body, 0)

    lanes = lax.iota(jnp.int32, L)

    def grp_body(g, carry):
        gbase = g * (L * L)
        acc = plsc.load_gather(q_v, [gbase + lanes * L])
        for d in range(1, L):
            acc = acc + plsc.load_gather(q_v, [gbase + lanes * L + d])
        out_v[pl.ds(g * L, L)] = acc
        return carry

    lax.fori_loop(0, BPW // L, grp_body, 0)

    pltpu.sync_copy(out_v, out_hbm.at[pl.ds(base, BPW)])


def kernel(i, j, node_emb):
    return _line_first_sc(i, j, node_emb)


# speedup vs baseline: 1.6327x; 1.6327x over previous
import functools

import jax
import jax.numpy as jnp
from jax import lax
from jax.experimental import pallas as pl
from jax.experimental.pallas import tpu as pltpu
from jax.experimental.pallas import tpu_sc as plsc

B = 16384
D = 64
L = 16
NC = 2
NS = 16
NW = NC * NS
BPW = B // NW
NCHUNK = BPW // L

_mesh = plsc.VectorSubcoreMesh(
    core_axis_name="c", subcore_axis_name="s", num_cores=NC, num_subcores=NS
)


@functools.partial(
    pl.kernel,
    out_type=jax.ShapeDtypeStruct((B,), jnp.float32),
    mesh=_mesh,
    compiler_params=pltpu.CompilerParams(needs_layout_passes=False),
    scratch_types=[
        pltpu.VMEM((BPW,), jnp.int32),
        pltpu.VMEM((BPW,), jnp.int32),
        pltpu.VMEM((L, D), jnp.float32),
        pltpu.VMEM((L, D), jnp.float32),
        pltpu.VMEM((BPW * L,), jnp.float32),
        pltpu.VMEM((BPW,), jnp.float32),
        pltpu.SemaphoreType.DMA,
    ],
)
def _line_first_sc(i_hbm, j_hbm, emb_hbm, out_hbm,
                   idxv_i, idxv_j, rows_i, rows_j, q_v, out_v, sem):
    wid = lax.axis_index("s") * NC + lax.axis_index("c")
    base = wid * BPW

    pltpu.sync_copy(i_hbm.at[pl.ds(base, BPW)], idxv_i)
    pltpu.sync_copy(j_hbm.at[pl.ds(base, BPW)], idxv_j)

    lanes = lax.iota(jnp.int32, L)
    zeros = jnp.zeros((L,), jnp.int32)

    def chunk_body(c, carry):
        vec_i = idxv_i[pl.ds(c * L, L)]
        vec_j = idxv_j[pl.ds(c * L, L)]
        copies = []
        for k in range(L):
            ri = lax.reduce_sum(jnp.where(lanes == k, vec_i, zeros), axes=(0,))
            copies.append(pltpu.async_copy(
                emb_hbm.at[ri], rows_i.at[k], sem))
            rj = lax.reduce_sum(jnp.where(lanes == k, vec_j, zeros), axes=(0,))
            copies.append(pltpu.async_copy(
                emb_hbm.at[rj], rows_j.at[k], sem))
        for cp in copies:
            cp.wait()
        for k in range(L):
            acc = rows_i[k, pl.ds(0, L)] * rows_j[k, pl.ds(0, L)]
            for cc in range(1, D // L):
                acc = acc + (rows_i[k, pl.ds(cc * L, L)]
                             * rows_j[k, pl.ds(cc * L, L)])
            q_v[pl.ds((c * L + k) * L, L)] = acc
        return carry

    lax.fori_loop(0, NCHUNK, chunk_body, 0)

    def grp_body(g, carry):
        gbase = g * (L * L)
        acc = plsc.load_gather(q_v, [gbase + lanes * L])
        for d in range(1, L):
            acc = acc + plsc.load_gather(q_v, [gbase + lanes * L + d])
        out_v[pl.ds(g * L, L)] = acc
        return carry

    lax.fori_loop(0, BPW // L, grp_body, 0)

    pltpu.sync_copy(out_v, out_hbm.at[pl.ds(base, BPW)])


def kernel(i, j, node_emb):
    return _line_first_sc(i, j, node_emb)


# speedup vs baseline: 1.6413x; 1.0053x over previous
import functools

import jax
import jax.numpy as jnp
from jax import lax
from jax.experimental import pallas as pl
from jax.experimental.pallas import tpu as pltpu
from jax.experimental.pallas import tpu_sc as plsc

B = 16384
D = 64
L = 16
NC = 2
NS = 16
NW = NC * NS
BPW = B // NW
NCHUNK = BPW // L
NBUF = 4
NROUND = NCHUNK // NBUF

_mesh = plsc.VectorSubcoreMesh(
    core_axis_name="c", subcore_axis_name="s", num_cores=NC, num_subcores=NS
)


@functools.partial(
    pl.kernel,
    out_type=jax.ShapeDtypeStruct((B,), jnp.float32),
    mesh=_mesh,
    compiler_params=pltpu.CompilerParams(needs_layout_passes=False),
    scratch_types=[
        pltpu.VMEM((BPW,), jnp.int32),
        pltpu.VMEM((BPW,), jnp.int32),
        pltpu.VMEM((NBUF, L, D), jnp.float32),
        pltpu.VMEM((NBUF, L, D), jnp.float32),
        pltpu.VMEM((BPW * L,), jnp.float32),
        pltpu.VMEM((BPW,), jnp.float32),
        pltpu.SemaphoreType.DMA,
        pltpu.SemaphoreType.DMA,
        pltpu.SemaphoreType.DMA,
        pltpu.SemaphoreType.DMA,
    ],
)
def _line_first_sc(i_hbm, j_hbm, emb_hbm, out_hbm,
                   idxv_i, idxv_j, rows_i, rows_j, q_v, out_v,
                   sem0, sem1, sem2, sem3):
    sems = [sem0, sem1, sem2, sem3]
    wid = lax.axis_index("s") * NC + lax.axis_index("c")
    base = wid * BPW

    pltpu.sync_copy(i_hbm.at[pl.ds(base, BPW)], idxv_i)
    pltpu.sync_copy(j_hbm.at[pl.ds(base, BPW)], idxv_j)

    lanes = lax.iota(jnp.int32, L)
    zeros = jnp.zeros((L,), jnp.int32)

    def fire(c, slot, sem):
        vec_i = idxv_i[pl.ds(c * L, L)]
        vec_j = idxv_j[pl.ds(c * L, L)]
        for k in range(L):
            ri = lax.reduce_sum(jnp.where(lanes == k, vec_i, zeros), axes=(0,))
            pltpu.async_copy(emb_hbm.at[ri], rows_i.at[slot, k], sem)
            rj = lax.reduce_sum(jnp.where(lanes == k, vec_j, zeros), axes=(0,))
            pltpu.async_copy(emb_hbm.at[rj], rows_j.at[slot, k], sem)

    def drain(slot, sem):
        for k in range(L):
            pltpu.make_async_copy(emb_hbm.at[0], rows_i.at[slot, k], sem).wait()
            pltpu.make_async_copy(emb_hbm.at[0], rows_j.at[slot, k], sem).wait()

    for b in range(NBUF):
        fire(b, b, sems[b])

    def round_body(g, carry):
        for b in range(NBUF):
            c = g * NBUF + b
            drain(b, sems[b])
            for k in range(L):
                acc = rows_i[b, k, pl.ds(0, L)] * rows_j[b, k, pl.ds(0, L)]
                for cc in range(1, D // L):
                    acc = acc + (rows_i[b, k, pl.ds(cc * L, L)]
                                 * rows_j[b, k, pl.ds(cc * L, L)])
                q_v[pl.ds((c * L + k) * L, L)] = acc

            @pl.when(c + NBUF < NCHUNK)
            def _():
                fire(c + NBUF, b, sems[b])
        return carry

    lax.fori_loop(0, NROUND, round_body, 0)

    def grp_body(g, carry):
        gbase = g * (L * L)
        acc = plsc.load_gather(q_v, [gbase + lanes * L])
        for d in range(1, L):
            acc = acc + plsc.load_gather(q_v, [gbase + lanes * L + d])
        out_v[pl.ds(g * L, L)] = acc
        return carry

    lax.fori_loop(0, BPW // L, grp_body, 0)

    pltpu.sync_copy(out_v, out_hbm.at[pl.ds(base, BPW)])


def kernel(i, j, node_emb):
    return _line_first_sc(i, j, node_emb)


# speedup vs baseline: 1.7165x; 1.0458x over previous
import functools

import jax
import jax.numpy as jnp
from jax import lax
from jax.experimental import pallas as pl
from jax.experimental.pallas import tpu as pltpu
from jax.experimental.pallas import tpu_sc as plsc

B = 16384
D = 64
L = 16
NC = 2
NS = 16
NW = NC * NS
BPW = B // NW
WAVE = 256
NWAVE = BPW // WAVE
WCHUNK = WAVE // L

_mesh = plsc.VectorSubcoreMesh(
    core_axis_name="c", subcore_axis_name="s", num_cores=NC, num_subcores=NS
)


@functools.partial(
    pl.kernel,
    out_type=jax.ShapeDtypeStruct((B,), jnp.float32),
    mesh=_mesh,
    compiler_params=pltpu.CompilerParams(needs_layout_passes=False),
    scratch_types=[
        pltpu.VMEM((BPW,), jnp.int32),
        pltpu.VMEM((BPW,), jnp.int32),
        pltpu.VMEM((WAVE, D), jnp.float32),
        pltpu.VMEM((WAVE, D), jnp.float32),
        pltpu.VMEM((BPW * L,), jnp.float32),
        pltpu.VMEM((BPW,), jnp.float32),
        pltpu.SemaphoreType.DMA,
    ],
)
def _line_first_sc(i_hbm, j_hbm, emb_hbm, out_hbm,
                   idxv_i, idxv_j, rows_i, rows_j, q_v, out_v, sem):
    wid = lax.axis_index("s") * NC + lax.axis_index("c")
    base = wid * BPW

    pltpu.sync_copy(i_hbm.at[pl.ds(base, BPW)], idxv_i)
    pltpu.sync_copy(j_hbm.at[pl.ds(base, BPW)], idxv_j)

    lanes = lax.iota(jnp.int32, L)
    zeros = jnp.zeros((L,), jnp.int32)

    def wave_body(w, carry):
        wb = w * WAVE

        def fire_body(c, carry2):
            vec_i = idxv_i[pl.ds(wb + c * L, L)]
            vec_j = idxv_j[pl.ds(wb + c * L, L)]
            for k in range(L):
                ri = lax.reduce_sum(
                    jnp.where(lanes == k, vec_i, zeros), axes=(0,))
                pltpu.async_copy(emb_hbm.at[ri], rows_i.at[c * L + k], sem)
                rj = lax.reduce_sum(
                    jnp.where(lanes == k, vec_j, zeros), axes=(0,))
                pltpu.async_copy(emb_hbm.at[rj], rows_j.at[c * L + k], sem)
            return carry2

        lax.fori_loop(0, WCHUNK, fire_body, 0)

        pltpu.make_async_copy(emb_hbm.at[pl.ds(0, WAVE)], rows_i, sem).wait()
        pltpu.make_async_copy(emb_hbm.at[pl.ds(0, WAVE)], rows_j, sem).wait()

        def compute_body(b, carry2):
            acc = rows_i[b, pl.ds(0, L)] * rows_j[b, pl.ds(0, L)]
            for cc in range(1, D // L):
                acc = acc + (rows_i[b, pl.ds(cc * L, L)]
                             * rows_j[b, pl.ds(cc * L, L)])
            q_v[pl.ds((wb + b) * L, L)] = acc
            return carry2

        lax.fori_loop(0, WAVE, compute_body, 0)
        return carry

    lax.fori_loop(0, NWAVE, wave_body, 0)

    def grp_body(g, carry):
        gbase = g * (L * L)
        acc = plsc.load_gather(q_v, [gbase + lanes * L])
        for d in range(1, L):
            acc = acc + plsc.load_gather(q_v, [gbase + lanes * L + d])
        out_v[pl.ds(g * L, L)] = acc
        return carry

    lax.fori_loop(0, BPW // L, grp_body, 0)

    pltpu.sync_copy(out_v, out_hbm.at[pl.ds(base, BPW)])


def kernel(i, j, node_emb):
    return _line_first_sc(i, j, node_emb)
